# baseline (speedup 1.0000x reference)
import functools

import jax
import jax.numpy as jnp
from jax import lax
from jax.experimental import pallas as pl
from jax.experimental.pallas import tpu as pltpu
from jax.experimental.pallas import tpu_sc as plsc

N = 50000
EMB = 64
ATTR = 16
HID = 256
E = 800000
B = 64

ACC_ROWS = 51200
E_PAD = 819200
IDX_ROWS = E_PAD // 128
R_BLK = 2048
GRID = ACC_ROWS // R_BLK

def _mesh():
    return plsc.VectorSubcoreMesh(core_axis_name="c", subcore_axis_name="s")


f32 = jnp.float32


def _deg_pass(dst_sh, dst_ss, dst_hh, ones_hbm, zeros_hbm):
    @functools.partial(
        pl.kernel,
        out_type=[jax.ShapeDtypeStruct((ACC_ROWS,), f32)] * 6,
        mesh=_mesh(),
        compiler_params=pltpu.CompilerParams(use_tc_tiling_on_sc=False),
        scratch_types=[
            pltpu.VMEM((8, 128), jnp.int32),
            pltpu.VMEM((128,), f32),
            pltpu.VMEM_SHARED((ACC_ROWS,), f32),
            pltpu.SemaphoreType.DMA,
        ],
    )
    def k(dsh, dss, dhh, ones_h, zf, o0, o1, o2, o3, o4, o5,
          idxd, ones_v, acc, ssem):
        cid = lax.axis_index("c")
        sid = lax.axis_index("s")
        wid = cid * 16 + sid
        pltpu.sync_copy(ones_h, ones_v)
        graphs = [dsh, dss, dhh]
        outs = [(o0, o1), (o2, o3), (o4, o5)]
        for g in range(3):
            pltpu.sync_copy(zf, acc.at[pl.ds(sid * 3200, 3200)])
            plsc.subcore_barrier()
            dstr = graphs[g]

            def grp(t, carry, dstr=dstr):
                r0 = wid * 200 + t * 8
                pltpu.sync_copy(dstr.at[pl.ds(r0, 8)], idxd)
                scs = [pltpu.async_copy(ones_v, acc.at[idxd.at[j]], ssem,
                                        add=True) for j in range(8)]
                for cp in scs:
                    cp.wait()
                return carry

            lax.fori_loop(0, 25, grp, 0)
            plsc.subcore_barrier()
            for c in (0, 1):
                @pl.when(cid == c)
                def _(out=outs[g][c]):
                    pltpu.sync_copy(acc.at[pl.ds(sid * 3200, 3200)],
                                    out.at[pl.ds(sid * 3200, 3200)])
            plsc.subcore_barrier()

    return k(dst_sh, dst_ss, dst_hh, ones_hbm, zeros_hbm)


def _scatter_pass(t0, t1, t2, t3, srcA, dstA, srcB, dstB, zeros_hbm):
    BLK_ROWS = 40
    NBLK = 400 // BLK_ROWS

    @functools.partial(
        pl.kernel,
        out_type=[jax.ShapeDtypeStruct((ACC_ROWS, 32), f32)] * 4,
        mesh=_mesh(),
        compiler_params=pltpu.CompilerParams(use_tc_tiling_on_sc=False),
        scratch_types=[
            pltpu.VMEM((BLK_ROWS, 128), jnp.int32),
            pltpu.VMEM((BLK_ROWS, 128), jnp.int32),
            pltpu.VMEM((4, 128, 32), f32),
            pltpu.VMEM_SHARED((ACC_ROWS, 32), f32),
            [pltpu.SemaphoreType.DMA] * 4,
            [pltpu.SemaphoreType.DMA] * 4,
        ],
    )
    def k(t0r, t1r, t2r, t3r, sA, dA, sB, dB, zf, o0, o1, o2, o3,
          idxs, idxd, rows, acc, gsems, ssems):
        cid = lax.axis_index("c")
        sid = lax.axis_index("s")
        cfg = {0: [(t0r, sA, dA, o0), (t1r, sA, dA, o1)],
               1: [(t2r, sB, dB, o2), (t3r, sB, dB, o3)]}
        for slot in range(2):
            pltpu.sync_copy(zf, acc.at[pl.ds(sid * 3200, 3200)])
            plsc.subcore_barrier()
            for c in (0, 1):
                tbl, src, dst, _o = cfg[c][slot]

                @pl.when(cid == c)
                def _(tbl=tbl, src=src, dst=dst):
                    def blk(b, carry):
                        r0 = sid * 400 + b * BLK_ROWS
                        pltpu.sync_copy(src.at[pl.ds(r0, BLK_ROWS)], idxs)
                        pltpu.sync_copy(dst.at[pl.ds(r0, BLK_ROWS)], idxd)
                        for j in range(2):
                            pltpu.async_copy(
                                tbl.at[idxs.at[j]], rows.at[j], gsems[j])

                        def ch(j4, c2):
                            for kk in range(4):
                                j = j4 * 4 + kk
                                spre = (kk + 2) % 4
                                pltpu.make_async_copy(
                                    tbl.at[idxs.at[j]], rows.at[kk],
                                    gsems[kk]).wait()
                                pltpu.async_copy(
                                    rows.at[kk], acc.at[idxd.at[j]],
                                    ssems[kk], add=True)

                                @pl.when(j >= 2)
                                def _(kk=kk, spre=spre, j=j):
                                    pltpu.make_async_copy(
                                        rows.at[spre],
                                        acc.at[idxd.at[j]],
                                        ssems[spre]).wait()

                                @pl.when(j + 2 < BLK_ROWS)
                                def _(j=j, spre=spre):
                                    pltpu.async_copy(
                                        tbl.at[idxs.at[j + 2]],
                                        rows.at[spre], gsems[spre])
                            return c2

                        lax.fori_loop(0, BLK_ROWS // 4, ch, 0)
                        for kk in (2, 3):
                            pltpu.make_async_copy(
                                rows.at[kk], acc.at[idxd.at[kk]],
                                ssems[kk]).wait()
                        return carry

                    lax.fori_loop(0, NBLK, blk, 0)
            plsc.subcore_barrier()
            for c in (0, 1):
                _t, _s, _d, out = cfg[c][slot]

                @pl.when(cid == c)
                def _(out=out):
                    pltpu.sync_copy(acc.at[pl.ds(sid * 3200, 3200)],
                                    out.at[pl.ds(sid * 3200, 3200)])
            plsc.subcore_barrier()

    return k(t0, t1, t2, t3, srcA, dstA, srcB, dstB, zeros_hbm)


def _row_spec(width):
    return pl.BlockSpec((R_BLK, width), lambda i: (i, 0))


def _full_spec(shape):
    return pl.BlockSpec(shape, lambda i: tuple(0 for _ in shape))


def _dense1(dis_sh, dis_ss, dis_hh, s_table, h_table,
            attrs, W0s, W0h, gcnW, kgW):
    def body(dsh_r, dss_r, dhh_r, s_ref, h_ref, a_ref,
             w0s, w0h, wg, wk, hA_o, hC_o, hD_o):
        dis_sh = dsh_r[...]
        dis_ss = dss_r[...]
        dis_hh = dhh_r[...]
        s = s_ref[...]
        h = h_ref[...]
        hA_o[...] = dis_sh * jnp.concatenate(
            [jnp.dot(s, w0s[...], preferred_element_type=f32),
             jnp.dot(h, w0h[...], preferred_element_type=f32)], axis=1)
        hC_o[...] = dis_ss * jnp.dot(s, wg[...], preferred_element_type=f32)
        hD_o[...] = dis_hh * (
            jnp.dot(h, wk[:EMB, :], preferred_element_type=f32)
            + jnp.dot(a_ref[...], wk[EMB:, :], preferred_element_type=f32))

    return pl.pallas_call(
        body,
        grid=(GRID,),
        in_specs=[_row_spec(128), _row_spec(EMB), _row_spec(EMB),
                  _row_spec(EMB), _row_spec(EMB),
                                    _row_spec(ATTR),
                                    _full_spec((EMB, EMB)),
                                    _full_spec((EMB, EMB)),
                                    _full_spec((EMB, EMB)),
                                    _full_spec((EMB + ATTR, EMB))],
        out_specs=[_row_spec(128), _row_spec(EMB), _row_spec(EMB)],
        out_shape=[jax.ShapeDtypeStruct((ACC_ROWS, 128), f32),
                   jax.ShapeDtypeStruct((ACC_ROWS, EMB), f32),
                   jax.ShapeDtypeStruct((ACC_ROWS, EMB), f32)],
    )(dis_sh, dis_ss, dis_hh, s_table, h_table, attrs,
      W0s, W0h, gcnW, kgW)


def _post1(accA, hA, dis_sh, b0cat, W1s, W1h):
    def body(acc_ref, hA_ref, dis_r, b0, w1s, w1h, l0_o, hB_o):
        dis = dis_r[...]
        l0 = jnp.tanh(dis * (acc_ref[...] + hA_ref[...]) + b0[...])
        l0_o[...] = l0
        hB_o[...] = dis * jnp.concatenate(
            [jnp.dot(l0[:, :EMB], w1s[...], preferred_element_type=f32),
             jnp.dot(l0[:, EMB:], w1h[...], preferred_element_type=f32)],
            axis=1)

    return pl.pallas_call(
        body,
        grid=(GRID,),
        in_specs=[_row_spec(128), _row_spec(128), _row_spec(128),
                  _full_spec((1, 128)), _full_spec((EMB, EMB)),
                  _full_spec((EMB, EMB))],
        out_specs=[_row_spec(128), _row_spec(128)],
        out_shape=[jax.ShapeDtypeStruct((ACC_ROWS, 128), f32),
                   jax.ShapeDtypeStruct((ACC_ROWS, 128), f32)],
    )(accA, hA, dis_sh, b0cat, W1s, W1h)


def _post2(accB, hB, l0, accC, hC, accD, hD, dis_sh_a, dis_ss_a, dis_hh_a,
           b1cat, gcn_b, kg_b, s_fu_W, s_fu_b, h_fu_W, h_fu_b,
           symptom):
    def body(accB_r, hB_r, l0_r, accC_r, hC_r, accD_r, hD_r,
             dsh_r, dss_r, dhh_r,
             b1, gb, kb, sfw, sfb, hfw, hfb, sym_r, hcomb_o, e0_o):
        i = pl.program_id(0)
        dis_sh = dsh_r[...]
        dis_ss = dss_r[...]
        dis_hh = dhh_r[...]
        l1 = dis_sh * (accB_r[...] + hB_r[...]) + b1[...]
        fused = 1.5 * l0_r[...] + 0.5 * l1
        s_sh = jnp.tanh(jnp.dot(fused[:, :EMB], sfw[...],
                                preferred_element_type=f32) + sfb[...])
        h_sh = jnp.tanh(jnp.dot(fused[:, EMB:], hfw[...],
                                preferred_element_type=f32) + hfb[...])
        s_ss = jnp.tanh(dis_ss * (accC_r[...] + hC_r[...]) + gb[...])
        h_hh = jnp.tanh(dis_hh * (accD_r[...] + hD_r[...]) + kb[...])
        s_comb = s_sh + s_ss
        hcomb_o[...] = h_sh + h_hh

        @pl.when(i == 0)
        def _():
            e0_o[...] = jnp.zeros((B, EMB), f32)

        e0_o[...] += jnp.dot(sym_r[...], s_comb, preferred_element_type=f32)

    return pl.pallas_call(
        body,
        grid=(GRID,),
        in_specs=[_row_spec(128), _row_spec(128), _row_spec(128),
                  _row_spec(EMB), _row_spec(EMB), _row_spec(EMB),
                  _row_spec(EMB),
                  _row_spec(128), _row_spec(EMB), _row_spec(EMB),
                  _full_spec((1, 128)), _full_spec((1, EMB)),
                  _full_spec((1, EMB)), _full_spec((EMB, EMB)),
                  _full_spec((1, EMB)), _full_spec((EMB, EMB)),
                  _full_spec((1, EMB)),
                  pl.BlockSpec((B, R_BLK), lambda i: (0, i))],
        out_specs=[_row_spec(EMB), _full_spec((B, EMB))],
        out_shape=[jax.ShapeDtypeStruct((ACC_ROWS, EMB), f32),
                   jax.ShapeDtypeStruct((B, EMB), f32)],
    )(accB, hB, l0, accC, hC, accD, hD, dis_sh_a, dis_ss_a, dis_hh_a,
      b1cat, gcn_b, kg_b, s_fu_W, s_fu_b, h_fu_W, h_fu_b,
      symptom)


def _logits(e0, mlp_W0, mlp_b0, mlp_W1, mlp_b1, h_comb):
    def body(e0_r, w0, b0, w1, b1, h_r, out_o):
        e = jnp.dot(
            jax.nn.relu(jnp.dot(e0_r[...], w0[...],
                                preferred_element_type=f32) + b0[...]),
            w1[...], preferred_element_type=f32) + b1[...]
        lg = lax.dot_general(e, h_r[...], (((1,), (1,)), ((), ())),
                             preferred_element_type=f32)
        out_o[...] = jax.nn.sigmoid(lg)

    return pl.pallas_call(
        body,
        grid=(GRID,),
        in_specs=[_full_spec((B, EMB)), _full_spec((EMB, HID)),
                  _full_spec((1, HID)), _full_spec((HID, EMB)),
                  _full_spec((1, EMB)), _row_spec(EMB)],
        out_specs=pl.BlockSpec((B, R_BLK), lambda i: (0, i)),
        out_shape=jax.ShapeDtypeStruct((B, ACC_ROWS), f32),
    )(e0, mlp_W0, mlp_b0, mlp_W1, mlp_b1, h_comb)


def kernel(symptom_set, herb_attributes, sh_graph, ss_graph, hh_graph,
           s_table, h_table,
           s_mu_W0, s_mu_b0, s_mu_W1, s_mu_b1,
           h_mu_W0, h_mu_b0, h_mu_W1, h_mu_b1,
           s_gcn_W, s_gcn_b, h_kg_W, h_kg_b,
           s_fu_W, s_fu_b, h_fu_W, h_fu_b,
           mlp_W0, mlp_b0, mlp_W1, mlp_b1):
    pad = E_PAD - E
    pad_src = ((jnp.arange(pad, dtype=jnp.int32) * 37) % N)
    pad_dst = N + (jnp.arange(pad, dtype=jnp.int32) % (ACC_ROWS - N))

    def prep(g):
        src = jnp.concatenate([g[0], pad_src]).reshape(IDX_ROWS, 128)
        dst = jnp.concatenate([g[1], pad_dst]).reshape(IDX_ROWS, 128)
        return src, dst

    src_sh, dst_sh = prep(sh_graph)
    src_ss, dst_ss = prep(ss_graph)
    src_hh, dst_hh = prep(hh_graph)

    ones_hbm = jnp.ones((128,), f32)
    zeros_deg = jnp.zeros((3200,), f32)
    zeros_feat = jnp.zeros((3200, 32), f32)

    npad = ACC_ROWS - N
    s_tab = jnp.pad(s_table, ((0, npad), (0, 0)))
    h_tab = jnp.pad(h_table, ((0, npad), (0, 0)))
    attr_p = jnp.pad(herb_attributes, ((0, npad), (0, 0)))
    sym_p = jnp.pad(symptom_set, ((0, 0), (0, npad)))

    degs = _deg_pass(dst_sh, dst_ss, dst_hh, ones_hbm, zeros_deg)

    def mkdis(pa, pb, w):
        return jnp.broadcast_to(lax.rsqrt(pa + pb + 1.0)[:, None],
                                (ACC_ROWS, w))

    dis_sh = mkdis(degs[0], degs[1], 128)
    dis_ss = mkdis(degs[2], degs[3], EMB)
    dis_hh = mkdis(degs[4], degs[5], EMB)

    hA, hC, hD = _dense1(dis_sh, dis_ss, dis_hh,
                         s_tab, h_tab, attr_p,
                         s_mu_W0, h_mu_W0, s_gcn_W, h_kg_W)

    split4 = lambda x: [x[:, 32 * i:32 * (i + 1)] for i in range(4)]
    cat4 = lambda xs: jnp.concatenate(xs, axis=1)

    hC0, hC1 = hC[:, :32], hC[:, 32:]
    hD0, hD1 = hD[:, :32], hD[:, 32:]
    accC0, accC1, accD0, accD1 = _scatter_pass(
        hC0, hC1, hD0, hD1, src_ss, dst_ss, src_hh, dst_hh, zeros_feat)
    a0, a1, a2, a3 = split4(hA)
    accA = cat4(_scatter_pass(a0, a1, a2, a3, src_sh, dst_sh, src_sh,
                              dst_sh, zeros_feat))

    b0cat = jnp.concatenate([s_mu_b0, h_mu_b0]).reshape(1, 128)
    l0, hB = _post1(accA, hA, dis_sh, b0cat, s_mu_W1, h_mu_W1)

    b0_, b1_, b2_, b3_ = split4(hB)
    accB = cat4(_scatter_pass(b0_, b1_, b2_, b3_, src_sh, dst_sh, src_sh,
                              dst_sh, zeros_feat))

    b1cat = jnp.concatenate([s_mu_b1, h_mu_b1]).reshape(1, 128)
    h_comb, e0 = _post2(
        accB, hB, l0, cat4([accC0, accC1]), hC, cat4([accD0, accD1]), hD,
        dis_sh, dis_ss, dis_hh,
        b1cat, s_gcn_b.reshape(1, EMB), h_kg_b.reshape(1, EMB),
        s_fu_W, s_fu_b.reshape(1, EMB), h_fu_W, h_fu_b.reshape(1, EMB),
        sym_p)

    out = _logits(e0, mlp_W0, mlp_b0.reshape(1, HID), mlp_W1,
                  mlp_b1.reshape(1, EMB), h_comb)
    return out[:, :N]


# speedup vs baseline: 1.0260x; 1.0260x over previous
import functools

import jax
import jax.numpy as jnp
from jax import lax
from jax.experimental import pallas as pl
from jax.experimental.pallas import tpu as pltpu
from jax.experimental.pallas import tpu_sc as plsc

N = 50000
EMB = 64
ATTR = 16
HID = 256
E = 800000
B = 64

ACC_ROWS = 51200
E_PAD = 819200
IDX_ROWS = E_PAD // 128
R_BLK = 2048
GRID = ACC_ROWS // R_BLK

def _mesh():
    return plsc.VectorSubcoreMesh(core_axis_name="c", subcore_axis_name="s")


f32 = jnp.float32


def _deg_pass(dst_sh, dst_ss, dst_hh, ones_hbm, zeros_hbm):
    @functools.partial(
        pl.kernel,
        out_type=[jax.ShapeDtypeStruct((ACC_ROWS,), f32)] * 6,
        mesh=_mesh(),
        compiler_params=pltpu.CompilerParams(use_tc_tiling_on_sc=False),
        scratch_types=[
            pltpu.VMEM((8, 128), jnp.int32),
            pltpu.VMEM((128,), f32),
            pltpu.VMEM_SHARED((ACC_ROWS,), f32),
            pltpu.SemaphoreType.DMA,
        ],
    )
    def k(dsh, dss, dhh, ones_h, zf, o0, o1, o2, o3, o4, o5,
          idxd, ones_v, acc, ssem):
        cid = lax.axis_index("c")
        sid = lax.axis_index("s")
        wid = cid * 16 + sid
        pltpu.sync_copy(ones_h, ones_v)
        graphs = [dsh, dss, dhh]
        outs = [(o0, o1), (o2, o3), (o4, o5)]
        for g in range(3):
            pltpu.sync_copy(zf, acc.at[pl.ds(sid * 3200, 3200)])
            plsc.subcore_barrier()
            dstr = graphs[g]

            def grp(t, carry, dstr=dstr):
                r0 = wid * 200 + t * 8
                pltpu.sync_copy(dstr.at[pl.ds(r0, 8)], idxd)
                scs = [pltpu.async_copy(ones_v, acc.at[idxd.at[j]], ssem,
                                        add=True) for j in range(8)]
                for cp in scs:
                    cp.wait()
                return carry

            lax.fori_loop(0, 25, grp, 0)
            plsc.subcore_barrier()
            for c in (0, 1):
                @pl.when(cid == c)
                def _(out=outs[g][c]):
                    pltpu.sync_copy(acc.at[pl.ds(sid * 3200, 3200)],
                                    out.at[pl.ds(sid * 3200, 3200)])
            plsc.subcore_barrier()

    return k(dst_sh, dst_ss, dst_hh, ones_hbm, zeros_hbm)


def _scatter_pass(t0, t1, t2, t3, srcA, dstA, srcB, dstB, zeros_hbm):
    BLK_ROWS = 40
    NBLK = 400 // BLK_ROWS

    @functools.partial(
        pl.kernel,
        out_type=[jax.ShapeDtypeStruct((ACC_ROWS, 32), f32)] * 4,
        mesh=_mesh(),
        compiler_params=pltpu.CompilerParams(use_tc_tiling_on_sc=False),
        scratch_types=[
            pltpu.VMEM((BLK_ROWS, 128), jnp.int32),
            pltpu.VMEM((4, 128), jnp.int32),
            pltpu.VMEM((4, 128, 32), f32),
            pltpu.VMEM_SHARED((ACC_ROWS, 32), f32),
            [pltpu.SemaphoreType.DMA] * 4,
            [pltpu.SemaphoreType.DMA] * 4,
        ],
    )
    def k(t0r, t1r, t2r, t3r, sA, dA, sB, dB, zf, o0, o1, o2, o3,
          idxs, idxd, rows, acc, gsems, ssems):
        cid = lax.axis_index("c")
        sid = lax.axis_index("s")
        cfg = {0: [(t0r, sA, dA, o0), (t1r, sA, dA, o1)],
               1: [(t2r, sB, dB, o2), (t3r, sB, dB, o3)]}
        for slot in range(2):
            pltpu.sync_copy(zf, acc.at[pl.ds(sid * 3200, 3200)])
            plsc.subcore_barrier()
            for c in (0, 1):
                tbl, src, dst, _o = cfg[c][slot]

                @pl.when(cid == c)
                def _(tbl=tbl, src=src, dst=dst):
                    def blk(b, carry):
                        r0 = sid * 400 + b * BLK_ROWS
                        pltpu.sync_copy(src.at[pl.ds(r0, BLK_ROWS)], idxs)
                        for j in range(2):
                            pltpu.async_copy(
                                tbl.at[idxs.at[j]], rows.at[j], gsems[j])
                            pltpu.async_copy(
                                dst.at[r0 + j], idxd.at[j], gsems[j])

                        def ch(j4, c2):
                            for kk in range(4):
                                j = j4 * 4 + kk
                                spre = (kk + 2) % 4
                                pltpu.make_async_copy(
                                    tbl.at[idxs.at[j]], rows.at[kk],
                                    gsems[kk]).wait()
                                pltpu.make_async_copy(
                                    dst.at[r0 + j], idxd.at[kk],
                                    gsems[kk]).wait()
                                @pl.when(j >= 1)
                                def _(kk=kk):
                                    kprev = (kk + 3) % 4
                                    pltpu.make_async_copy(
                                        rows.at[kprev],
                                        acc.at[idxd.at[kprev]],
                                        ssems[kprev]).wait()

                                pltpu.async_copy(
                                    rows.at[kk], acc.at[idxd.at[kk]],
                                    ssems[kk], add=True)

                                @pl.when(j + 2 < BLK_ROWS)
                                def _(j=j, spre=spre):
                                    pltpu.async_copy(
                                        tbl.at[idxs.at[j + 2]],
                                        rows.at[spre], gsems[spre])
                                    pltpu.async_copy(
                                        dst.at[r0 + j + 2], idxd.at[spre],
                                        gsems[spre])
                            return c2

                        lax.fori_loop(0, BLK_ROWS // 4, ch, 0)
                        pltpu.make_async_copy(
                            rows.at[3], acc.at[idxd.at[3]],
                            ssems[3]).wait()
                        return carry

                    lax.fori_loop(0, NBLK, blk, 0)
            plsc.subcore_barrier()
            for c in (0, 1):
                _t, _s, _d, out = cfg[c][slot]

                @pl.when(cid == c)
                def _(out=out):
                    pltpu.sync_copy(acc.at[pl.ds(sid * 3200, 3200)],
                                    out.at[pl.ds(sid * 3200, 3200)])
            plsc.subcore_barrier()

    return k(t0, t1, t2, t3, srcA, dstA, srcB, dstB, zeros_hbm)


def _row_spec(width):
    return pl.BlockSpec((R_BLK, width), lambda i: (i, 0))


def _full_spec(shape):
    return pl.BlockSpec(shape, lambda i: tuple(0 for _ in shape))


def _dense1(dis_sh, dis_ss, dis_hh, s_table, h_table,
            attrs, W0s, W0h, gcnW, kgW):
    def body(dsh_r, dss_r, dhh_r, s_ref, h_ref, a_ref,
             w0s, w0h, wg, wk, hA_o, hC_o, hD_o):
        dis_sh = dsh_r[...]
        dis_ss = dss_r[...]
        dis_hh = dhh_r[...]
        s = s_ref[...]
        h = h_ref[...]
        hA_o[...] = dis_sh * jnp.concatenate(
            [jnp.dot(s, w0s[...], preferred_element_type=f32),
             jnp.dot(h, w0h[...], preferred_element_type=f32)], axis=1)
        hC_o[...] = dis_ss * jnp.dot(s, wg[...], preferred_element_type=f32)
        hD_o[...] = dis_hh * (
            jnp.dot(h, wk[:EMB, :], preferred_element_type=f32)
            + jnp.dot(a_ref[...], wk[EMB:, :], preferred_element_type=f32))

    return pl.pallas_call(
        body,
        grid=(GRID,),
        in_specs=[_row_spec(128), _row_spec(EMB), _row_spec(EMB),
                  _row_spec(EMB), _row_spec(EMB),
                                    _row_spec(ATTR),
                                    _full_spec((EMB, EMB)),
                                    _full_spec((EMB, EMB)),
                                    _full_spec((EMB, EMB)),
                                    _full_spec((EMB + ATTR, EMB))],
        out_specs=[_row_spec(128), _row_spec(EMB), _row_spec(EMB)],
        out_shape=[jax.ShapeDtypeStruct((ACC_ROWS, 128), f32),
                   jax.ShapeDtypeStruct((ACC_ROWS, EMB), f32),
                   jax.ShapeDtypeStruct((ACC_ROWS, EMB), f32)],
    )(dis_sh, dis_ss, dis_hh, s_table, h_table, attrs,
      W0s, W0h, gcnW, kgW)


def _post1(accA, hA, dis_sh, b0cat, W1s, W1h):
    def body(acc_ref, hA_ref, dis_r, b0, w1s, w1h, l0_o, hB_o):
        dis = dis_r[...]
        l0 = jnp.tanh(dis * (acc_ref[...] + hA_ref[...]) + b0[...])
        l0_o[...] = l0
        hB_o[...] = dis * jnp.concatenate(
            [jnp.dot(l0[:, :EMB], w1s[...], preferred_element_type=f32),
             jnp.dot(l0[:, EMB:], w1h[...], preferred_element_type=f32)],
            axis=1)

    return pl.pallas_call(
        body,
        grid=(GRID,),
        in_specs=[_row_spec(128), _row_spec(128), _row_spec(128),
                  _full_spec((1, 128)), _full_spec((EMB, EMB)),
                  _full_spec((EMB, EMB))],
        out_specs=[_row_spec(128), _row_spec(128)],
        out_shape=[jax.ShapeDtypeStruct((ACC_ROWS, 128), f32),
                   jax.ShapeDtypeStruct((ACC_ROWS, 128), f32)],
    )(accA, hA, dis_sh, b0cat, W1s, W1h)


def _post2(accB, hB, l0, accC, hC, accD, hD, dis_sh_a, dis_ss_a, dis_hh_a,
           b1cat, gcn_b, kg_b, s_fu_W, s_fu_b, h_fu_W, h_fu_b,
           symptom):
    def body(accB_r, hB_r, l0_r, accC_r, hC_r, accD_r, hD_r,
             dsh_r, dss_r, dhh_r,
             b1, gb, kb, sfw, sfb, hfw, hfb, sym_r, hcomb_o, e0_o):
        i = pl.program_id(0)
        dis_sh = dsh_r[...]
        dis_ss = dss_r[...]
        dis_hh = dhh_r[...]
        l1 = dis_sh * (accB_r[...] + hB_r[...]) + b1[...]
        fused = 1.5 * l0_r[...] + 0.5 * l1
        s_sh = jnp.tanh(jnp.dot(fused[:, :EMB], sfw[...],
                                preferred_element_type=f32) + sfb[...])
        h_sh = jnp.tanh(jnp.dot(fused[:, EMB:], hfw[...],
                                preferred_element_type=f32) + hfb[...])
        s_ss = jnp.tanh(dis_ss * (accC_r[...] + hC_r[...]) + gb[...])
        h_hh = jnp.tanh(dis_hh * (accD_r[...] + hD_r[...]) + kb[...])
        s_comb = s_sh + s_ss
        hcomb_o[...] = h_sh + h_hh

        @pl.when(i == 0)
        def _():
            e0_o[...] = jnp.zeros((B, EMB), f32)

        e0_o[...] += jnp.dot(sym_r[...], s_comb, preferred_element_type=f32)

    return pl.pallas_call(
        body,
        grid=(GRID,),
        in_specs=[_row_spec(128), _row_spec(128), _row_spec(128),
                  _row_spec(EMB), _row_spec(EMB), _row_spec(EMB),
                  _row_spec(EMB),
                  _row_spec(128), _row_spec(EMB), _row_spec(EMB),
                  _full_spec((1, 128)), _full_spec((1, EMB)),
                  _full_spec((1, EMB)), _full_spec((EMB, EMB)),
                  _full_spec((1, EMB)), _full_spec((EMB, EMB)),
                  _full_spec((1, EMB)),
                  pl.BlockSpec((B, R_BLK), lambda i: (0, i))],
        out_specs=[_row_spec(EMB), _full_spec((B, EMB))],
        out_shape=[jax.ShapeDtypeStruct((ACC_ROWS, EMB), f32),
                   jax.ShapeDtypeStruct((B, EMB), f32)],
    )(accB, hB, l0, accC, hC, accD, hD, dis_sh_a, dis_ss_a, dis_hh_a,
      b1cat, gcn_b, kg_b, s_fu_W, s_fu_b, h_fu_W, h_fu_b,
      symptom)


def _logits(e0, mlp_W0, mlp_b0, mlp_W1, mlp_b1, h_comb):
    def body(e0_r, w0, b0, w1, b1, h_r, out_o):
        e = jnp.dot(
            jax.nn.relu(jnp.dot(e0_r[...], w0[...],
                                preferred_element_type=f32) + b0[...]),
            w1[...], preferred_element_type=f32) + b1[...]
        lg = lax.dot_general(e, h_r[...], (((1,), (1,)), ((), ())),
                             preferred_element_type=f32)
        out_o[...] = jax.nn.sigmoid(lg)

    return pl.pallas_call(
        body,
        grid=(GRID,),
        in_specs=[_full_spec((B, EMB)), _full_spec((EMB, HID)),
                  _full_spec((1, HID)), _full_spec((HID, EMB)),
                  _full_spec((1, EMB)), _row_spec(EMB)],
        out_specs=pl.BlockSpec((B, R_BLK), lambda i: (0, i)),
        out_shape=jax.ShapeDtypeStruct((B, ACC_ROWS), f32),
    )(e0, mlp_W0, mlp_b0, mlp_W1, mlp_b1, h_comb)


def kernel(symptom_set, herb_attributes, sh_graph, ss_graph, hh_graph,
           s_table, h_table,
           s_mu_W0, s_mu_b0, s_mu_W1, s_mu_b1,
           h_mu_W0, h_mu_b0, h_mu_W1, h_mu_b1,
           s_gcn_W, s_gcn_b, h_kg_W, h_kg_b,
           s_fu_W, s_fu_b, h_fu_W, h_fu_b,
           mlp_W0, mlp_b0, mlp_W1, mlp_b1):
    pad = E_PAD - E
    pad_src = ((jnp.arange(pad, dtype=jnp.int32) * 37) % N)
    pad_dst = N + (jnp.arange(pad, dtype=jnp.int32) % (ACC_ROWS - N))

    def prep(g):
        src = jnp.concatenate([g[0], pad_src]).reshape(IDX_ROWS, 128)
        dst = jnp.concatenate([g[1], pad_dst]).reshape(IDX_ROWS, 128)
        return src, dst

    src_sh, dst_sh = prep(sh_graph)
    src_ss, dst_ss = prep(ss_graph)
    src_hh, dst_hh = prep(hh_graph)

    ones_hbm = jnp.ones((128,), f32)
    zeros_deg = jnp.zeros((3200,), f32)
    zeros_feat = jnp.zeros((3200, 32), f32)

    npad = ACC_ROWS - N
    s_tab = jnp.pad(s_table, ((0, npad), (0, 0)))
    h_tab = jnp.pad(h_table, ((0, npad), (0, 0)))
    attr_p = jnp.pad(herb_attributes, ((0, npad), (0, 0)))
    sym_p = jnp.pad(symptom_set, ((0, 0), (0, npad)))

    degs = _deg_pass(dst_sh, dst_ss, dst_hh, ones_hbm, zeros_deg)

    def mkdis(pa, pb, w):
        return jnp.broadcast_to(lax.rsqrt(pa + pb + 1.0)[:, None],
                                (ACC_ROWS, w))

    dis_sh = mkdis(degs[0], degs[1], 128)
    dis_ss = mkdis(degs[2], degs[3], EMB)
    dis_hh = mkdis(degs[4], degs[5], EMB)

    hA, hC, hD = _dense1(dis_sh, dis_ss, dis_hh,
                         s_tab, h_tab, attr_p,
                         s_mu_W0, h_mu_W0, s_gcn_W, h_kg_W)

    split4 = lambda x: [x[:, 32 * i:32 * (i + 1)] for i in range(4)]
    cat4 = lambda xs: jnp.concatenate(xs, axis=1)

    hC0, hC1 = hC[:, :32], hC[:, 32:]
    hD0, hD1 = hD[:, :32], hD[:, 32:]
    accC0, accC1, accD0, accD1 = _scatter_pass(
        hC0, hC1, hD0, hD1, src_ss, dst_ss, src_hh, dst_hh, zeros_feat)
    a0, a1, a2, a3 = split4(hA)
    accA = cat4(_scatter_pass(a0, a1, a2, a3, src_sh, dst_sh, src_sh,
                              dst_sh, zeros_feat))

    b0cat = jnp.concatenate([s_mu_b0, h_mu_b0]).reshape(1, 128)
    l0, hB = _post1(accA, hA, dis_sh, b0cat, s_mu_W1, h_mu_W1)

    b0_, b1_, b2_, b3_ = split4(hB)
    accB = cat4(_scatter_pass(b0_, b1_, b2_, b3_, src_sh, dst_sh, src_sh,
                              dst_sh, zeros_feat))

    b1cat = jnp.concatenate([s_mu_b1, h_mu_b1]).reshape(1, 128)
    h_comb, e0 = _post2(
        accB, hB, l0, cat4([accC0, accC1]), hC, cat4([accD0, accD1]), hD,
        dis_sh, dis_ss, dis_hh,
        b1cat, s_gcn_b.reshape(1, EMB), h_kg_b.reshape(1, EMB),
        s_fu_W, s_fu_b.reshape(1, EMB), h_fu_W, h_fu_b.reshape(1, EMB),
        sym_p)

    out = _logits(e0, mlp_W0, mlp_b0.reshape(1, HID), mlp_W1,
                  mlp_b1.reshape(1, EMB), h_comb)
    return out[:, :N]
